# baseline (speedup 1.0000x reference)
import jax
import jax.numpy as jnp
from jax import lax
from jax.experimental import pallas as pl
from jax.experimental.pallas import tpu as pltpu
from jax.experimental.pallas import tpu_sc as plsc

D = 64
S = 32
NQ = S * S
ROWS = NQ + 1
MAXR = 14
NC = 2
NS = 16
L = 16
DCH = 16


def _sc_gather(v_hbm, h_hbm, vrepf_hbm, hrepf_hbm, vtab, htab, vb0, vb1,
               hb0, hb1, sem):
    w = lax.axis_index("s") * NC + lax.axis_index("c")

    pltpu.sync_copy(v_hbm, vtab)
    pltpu.sync_copy(h_hbm, htab)

    t_lo = lax.iota(jnp.int32, L)
    hrow_lo = jnp.clip(t_lo - w, -MAXR, MAXR) + MAXR + 1
    hrow_hi = jnp.clip(t_lo + L - w, -MAXR, MAXR) + MAXR + 1
    zero16 = jnp.zeros((L,), jnp.int32)
    lane0 = t_lo == 0

    vbufs = (vb0, vb1)
    hbufs = (hb0, hb1)
    handles = [None, None]
    for chunk in range(D // DCH):
        vb = vbufs[chunk % 2]
        hb = hbufs[chunk % 2]
        if handles[chunk % 2] is not None:
            for h in handles[chunk % 2]:
                h.wait()

        def _drow(dloc, carry, chunk=chunk, vb=vb, hb=hb):
            d = chunk * DCH + dloc
            dcol = jnp.full((L,), d, jnp.int32)
            hv_lo = plsc.load_gather(htab, [hrow_lo, dcol])
            hv_hi = plsc.load_gather(htab, [hrow_hi, dcol])
            cv = (plsc.load_gather(vtab, [zero16, dcol])
                  + plsc.load_gather(htab, [zero16, dcol]))
            drow = jnp.full((L,), dloc, jnp.int32)
            plsc.store_scatter(hb, [drow, zero16], cv, mask=lane0)
            plsc.store_scatter(vb, [drow, zero16],
                               jnp.zeros((L,), jnp.float32), mask=lane0)
            for b in range(S):
                vidx = jnp.clip(b - w, -MAXR, MAXR) + MAXR + 1
                vrow = jnp.full((L,), vidx, jnp.int32)
                vv = plsc.load_gather(vtab, [vrow, dcol])
                vb[dloc, pl.ds(1 + S * b, L)] = vv
                vb[dloc, pl.ds(1 + S * b + L, L)] = vv
                hb[dloc, pl.ds(1 + S * b, L)] = hv_lo
                hb[dloc, pl.ds(1 + S * b + L, L)] = hv_hi
            return carry

        lax.fori_loop(0, DCH, _drow, 0)
        handles[chunk % 2] = [
            pltpu.async_copy(
                vb, vrepf_hbm.at[w, pl.ds(chunk * DCH, DCH)], sem),
            pltpu.async_copy(
                hb, hrepf_hbm.at[w, pl.ds(chunk * DCH, DCH)], sem),
        ]

    for hs in handles:
        if hs is not None:
            for h in hs:
                h.wait()


def _tc_fanout(vrepf_ref, hrepf_ref, out_ref):
    g = pl.program_id(0)
    vwin = vrepf_ref[jnp.minimum(g, S - 1)]
    vwin0 = vrepf_ref[jnp.maximum(g, 1) - 1]
    hp31 = hrepf_ref[S - 1]

    const_row = jnp.broadcast_to(hp31[:, 0:1], (D, ROWS))
    row0 = jnp.where(g == 0, const_row, vwin0 + hp31)
    out_ref[0, :, :] = row0

    for r in range(1, S):
        out_ref[r, :, :] = vwin + hrepf_ref[r - 1]


@jax.jit
def _expand(v, h):
    vp = jnp.zeros((S, D), jnp.float32).at[: 2 * MAXR + 2].set(v)
    hp = jnp.zeros((S, D), jnp.float32).at[: 2 * MAXR + 2].set(h)

    mesh = plsc.VectorSubcoreMesh(core_axis_name="c", subcore_axis_name="s")
    vrepf, hrepf = pl.kernel(
        _sc_gather,
        out_type=(
            jax.ShapeDtypeStruct((S, D, ROWS), jnp.float32),
            jax.ShapeDtypeStruct((S, D, ROWS), jnp.float32),
        ),
        mesh=mesh,
        compiler_params=pltpu.CompilerParams(
            use_tc_tiling_on_sc=False, needs_layout_passes=False),
        scratch_types=[
            pltpu.VMEM((S, D), jnp.float32),
            pltpu.VMEM((S, D), jnp.float32),
            pltpu.VMEM((DCH, ROWS), jnp.float32),
            pltpu.VMEM((DCH, ROWS), jnp.float32),
            pltpu.VMEM((DCH, ROWS), jnp.float32),
            pltpu.VMEM((DCH, ROWS), jnp.float32),
            pltpu.SemaphoreType.DMA,
        ],
    )(vp, hp)

    out_t = pl.pallas_call(
        _tc_fanout,
        out_shape=jax.ShapeDtypeStruct((ROWS, D, ROWS), jnp.float32),
        grid=(S + 1,),
        in_specs=[
            pl.BlockSpec((S, D, ROWS), lambda g: (0, 0, 0)),
            pl.BlockSpec((S, D, ROWS), lambda g: (0, 0, 0)),
        ],
        out_specs=pl.BlockSpec((S, D, ROWS), lambda g: (g, 0, 0)),
    )(vrepf, hrepf)

    return jnp.transpose(out_t, (0, 2, 1))


def kernel(embeddings_table_v, embeddings_table_h, length_q, length_k):
    del length_q, length_k
    return _expand(embeddings_table_v, embeddings_table_h)
